# Initial kernel scaffold; baseline (speedup 1.0000x reference)
import jax
import jax.numpy as jnp
from jax.experimental import pallas as pl


def kernel(x, eps, se_w1, se_b1, se_w2, se_b2, bn_ca_gamma, bn_ca_beta, bn_ca_mean, bn_ca_var, ta_w1, ta_b1, ta_w2, ta_b2, ta_w3, ta_b3, ta_w4, ta_b4, bn_ta_gamma, bn_ta_beta, bn_ta_mean, bn_ta_var):
    raise NotImplementedError("write your pallas kernel here")


# speedup vs baseline: 1.5616x; 1.5616x over previous
import jax
import jax.numpy as jnp
from jax.experimental import pallas as pl
from jax.experimental.pallas import tpu as pltpu


def _bam_fused_kernel(x_ref, sew1t_ref, seb1_ref, sew2t_ref, seb2_ref,
                      cas_ref, cash_ref, w8_ref, tac_ref, o_ref):
    x = x_ref[0]
    L = x.shape[1]

    hi = jax.lax.Precision.HIGHEST
    y = jnp.mean(x, axis=1, keepdims=True)
    h = jnp.dot(sew1t_ref[...], y, preferred_element_type=jnp.float32,
                precision=hi)
    h = jnp.maximum(h + seb1_ref[...], 0.0)
    s = jnp.dot(sew2t_ref[...], h, preferred_element_type=jnp.float32,
                precision=hi)
    s = jax.nn.sigmoid(s + seb2_ref[...])
    gate = s * cas_ref[...]
    shift = cash_ref[...]

    a = jnp.dot(w8_ref[...], x, preferred_element_type=jnp.float32,
                precision=hi)

    def shr(r, n):
        return jnp.concatenate(
            [jnp.zeros((1, n), jnp.float32), r[:, :L - n]], axis=1)

    def shl(r, n):
        return jnp.concatenate(
            [r[:, n:], jnp.zeros((1, n), jnp.float32)], axis=1)

    ta = (tac_ref[...] + shr(a[0:1], 8) + shr(a[1:2], 4) + a[2:3]
          + shl(a[3:4], 4) + shl(a[4:5], 8))
    idx = jax.lax.broadcasted_iota(jnp.int32, (1, L), 1)
    ta = ta - jnp.where(idx < 4, a[5:6], 0.0)
    ta = ta - jnp.where(idx >= L - 4, a[6:7], 0.0)

    ca = gate * x + shift
    o_ref[0] = ca * (ta * x) + x


def kernel(x, eps, se_w1, se_b1, se_w2, se_b2, bn_ca_gamma, bn_ca_beta,
           bn_ca_mean, bn_ca_var, ta_w1, ta_b1, ta_w2, ta_b2, ta_w3, ta_b3,
           ta_w4, ta_b4, bn_ta_gamma, bn_ta_beta, bn_ta_mean, bn_ta_var):
    B, C, L = x.shape
    Cr = se_w1.shape[1]

    bn_scale = bn_ca_gamma * jax.lax.rsqrt(bn_ca_var + eps)
    cas = bn_scale[:, None]
    cash = (bn_ca_beta - bn_ca_mean * bn_scale)[:, None]

    bnt_scale = bn_ta_gamma * jax.lax.rsqrt(bn_ta_var + eps)
    w4e = ta_w4 * bnt_scale
    b4e = ta_b4 * bnt_scale + bn_ta_beta - bn_ta_mean * bnt_scale

    w2t = jnp.transpose(ta_w2, (2, 0, 1))
    w3t = jnp.transpose(ta_w3, (2, 0, 1))
    v = jnp.einsum('c,jcd->jd', w4e[0], w3t)
    m = jnp.einsum('jc,icd->jid', v, w2t)
    u = jnp.stack([m[0, 0], m[0, 1] + m[1, 0],
                   m[0, 2] + m[1, 1] + m[2, 0],
                   m[1, 2] + m[2, 1], m[2, 2]])
    w8 = jnp.concatenate([u @ ta_w1,
                          (m[0, 2] @ ta_w1)[None],
                          (m[2, 0] @ ta_w1)[None],
                          jnp.zeros((1, C), jnp.float32)], axis=0)

    t = jnp.arange(L)
    const = jnp.full((L,), b4e[0] + w4e[0] @ ta_b3, jnp.float32)
    for j in range(3):
        o = 4 * (j - 1)
        const += jnp.where((t + o >= 0) & (t + o < L), v[j] @ ta_b2, 0.0)
    for sft in range(5):
        o = 4 * (sft - 2)
        const += jnp.where((t + o >= 0) & (t + o < L), u[sft] @ ta_b1, 0.0)
    const -= jnp.where(t < 4, m[0, 2] @ ta_b1, 0.0)
    const -= jnp.where(t >= L - 4, m[2, 0] @ ta_b1, 0.0)
    tac = const[None, :]

    sew1t = se_w1.T
    sew2t = se_w2.T
    seb1 = se_b1[:, None]
    seb2 = se_b2[:, None]

    nbytes = x.size * x.dtype.itemsize
    cost = pl.CostEstimate(
        flops=2 * B * 8 * C * L + 7 * B * C * L,
        transcendentals=B * C,
        bytes_accessed=2 * nbytes,
    )
    return pl.pallas_call(
        _bam_fused_kernel,
        out_shape=jax.ShapeDtypeStruct((B, C, L), x.dtype),
        grid=(B,),
        in_specs=[
            pl.BlockSpec((1, C, L), lambda b: (b, 0, 0)),
            pl.BlockSpec((Cr, C), lambda b: (0, 0)),
            pl.BlockSpec((Cr, 1), lambda b: (0, 0)),
            pl.BlockSpec((C, Cr), lambda b: (0, 0)),
            pl.BlockSpec((C, 1), lambda b: (0, 0)),
            pl.BlockSpec((C, 1), lambda b: (0, 0)),
            pl.BlockSpec((C, 1), lambda b: (0, 0)),
            pl.BlockSpec((8, C), lambda b: (0, 0)),
            pl.BlockSpec((1, L), lambda b: (0, 0)),
        ],
        out_specs=pl.BlockSpec((1, C, L), lambda b: (b, 0, 0)),
        compiler_params=pltpu.CompilerParams(
            dimension_semantics=("parallel",)),
        cost_estimate=cost,
    )(x, sew1t, seb1, sew2t, seb2, cas, cash, w8, tac)


# speedup vs baseline: 1.9347x; 1.2389x over previous
import jax
import jax.numpy as jnp
from jax.experimental import pallas as pl
from jax.experimental.pallas import tpu as pltpu


def _bam_fused_kernel(x_ref, sew1t_ref, seb1_ref, sew2t_ref, seb2_ref,
                      cas_ref, cash_ref, w8_ref, tac_ref, o_ref):
    x = x_ref[0]
    L = x.shape[1]

    y = jnp.mean(x, axis=1, keepdims=True)
    h = jnp.dot(sew1t_ref[...], y, preferred_element_type=jnp.float32)
    h = jnp.maximum(h + seb1_ref[...], 0.0)
    s = jnp.dot(sew2t_ref[...], h, preferred_element_type=jnp.float32)
    s = jax.nn.sigmoid(s + seb2_ref[...])
    gate = s * cas_ref[...]
    shift = cash_ref[...]

    a = jnp.dot(w8_ref[...], x, preferred_element_type=jnp.float32)

    def shr(r, n):
        return jnp.concatenate(
            [jnp.zeros((1, n), jnp.float32), r[:, :L - n]], axis=1)

    def shl(r, n):
        return jnp.concatenate(
            [r[:, n:], jnp.zeros((1, n), jnp.float32)], axis=1)

    ta = (tac_ref[...] + shr(a[0:1], 8) + shr(a[1:2], 4) + a[2:3]
          + shl(a[3:4], 4) + shl(a[4:5], 8))
    idx = jax.lax.broadcasted_iota(jnp.int32, (1, L), 1)
    ta = ta - jnp.where(idx < 4, a[5:6], 0.0)
    ta = ta - jnp.where(idx >= L - 4, a[6:7], 0.0)

    ca = gate * x + shift
    o_ref[0] = ca * (ta * x) + x


def kernel(x, eps, se_w1, se_b1, se_w2, se_b2, bn_ca_gamma, bn_ca_beta,
           bn_ca_mean, bn_ca_var, ta_w1, ta_b1, ta_w2, ta_b2, ta_w3, ta_b3,
           ta_w4, ta_b4, bn_ta_gamma, bn_ta_beta, bn_ta_mean, bn_ta_var):
    B, C, L = x.shape
    Cr = se_w1.shape[1]

    bn_scale = bn_ca_gamma * jax.lax.rsqrt(bn_ca_var + eps)
    cas = bn_scale[:, None]
    cash = (bn_ca_beta - bn_ca_mean * bn_scale)[:, None]

    bnt_scale = bn_ta_gamma * jax.lax.rsqrt(bn_ta_var + eps)
    w4e = ta_w4 * bnt_scale
    b4e = ta_b4 * bnt_scale + bn_ta_beta - bn_ta_mean * bnt_scale

    w2t = jnp.transpose(ta_w2, (2, 0, 1))
    w3t = jnp.transpose(ta_w3, (2, 0, 1))
    v = jnp.einsum('c,jcd->jd', w4e[0], w3t)
    m = jnp.einsum('jc,icd->jid', v, w2t)
    u = jnp.stack([m[0, 0], m[0, 1] + m[1, 0],
                   m[0, 2] + m[1, 1] + m[2, 0],
                   m[1, 2] + m[2, 1], m[2, 2]])
    w8 = jnp.concatenate([u @ ta_w1,
                          (m[0, 2] @ ta_w1)[None],
                          (m[2, 0] @ ta_w1)[None],
                          jnp.zeros((1, C), jnp.float32)], axis=0)

    t = jnp.arange(L)
    const = jnp.full((L,), b4e[0] + w4e[0] @ ta_b3, jnp.float32)
    for j in range(3):
        o = 4 * (j - 1)
        const += jnp.where((t + o >= 0) & (t + o < L), v[j] @ ta_b2, 0.0)
    for sft in range(5):
        o = 4 * (sft - 2)
        const += jnp.where((t + o >= 0) & (t + o < L), u[sft] @ ta_b1, 0.0)
    const -= jnp.where(t < 4, m[0, 2] @ ta_b1, 0.0)
    const -= jnp.where(t >= L - 4, m[2, 0] @ ta_b1, 0.0)
    tac = const[None, :]

    sew1t = se_w1.T
    sew2t = se_w2.T
    seb1 = se_b1[:, None]
    seb2 = se_b2[:, None]

    nbytes = x.size * x.dtype.itemsize
    cost = pl.CostEstimate(
        flops=2 * B * 8 * C * L + 7 * B * C * L,
        transcendentals=B * C,
        bytes_accessed=2 * nbytes,
    )
    return pl.pallas_call(
        _bam_fused_kernel,
        out_shape=jax.ShapeDtypeStruct((B, C, L), x.dtype),
        grid=(B,),
        in_specs=[
            pl.BlockSpec((1, C, L), lambda b: (b, 0, 0)),
            pl.BlockSpec((Cr, C), lambda b: (0, 0)),
            pl.BlockSpec((Cr, 1), lambda b: (0, 0)),
            pl.BlockSpec((C, Cr), lambda b: (0, 0)),
            pl.BlockSpec((C, 1), lambda b: (0, 0)),
            pl.BlockSpec((C, 1), lambda b: (0, 0)),
            pl.BlockSpec((C, 1), lambda b: (0, 0)),
            pl.BlockSpec((8, C), lambda b: (0, 0)),
            pl.BlockSpec((1, L), lambda b: (0, 0)),
        ],
        out_specs=pl.BlockSpec((1, C, L), lambda b: (b, 0, 0)),
        compiler_params=pltpu.CompilerParams(
            dimension_semantics=("parallel",)),
        cost_estimate=cost,
    )(x, sew1t, seb1, sew2t, seb2, cas, cash, w8, tac)


# speedup vs baseline: 2.1910x; 1.1325x over previous
import jax
import jax.numpy as jnp
from jax.experimental import pallas as pl
from jax.experimental.pallas import tpu as pltpu


_ROWS = 2


def _bam_fused_kernel(x_ref, sew1t_ref, seb1_ref, sew2t_ref, seb2_ref,
                      cas_ref, cash_ref, w8_ref, tac_ref, o_ref):
    L = x_ref.shape[2]

    def shr(r, n):
        return jnp.concatenate(
            [jnp.zeros((1, n), jnp.float32), r[:, :L - n]], axis=1)

    def shl(r, n):
        return jnp.concatenate(
            [r[:, n:], jnp.zeros((1, n), jnp.float32)], axis=1)

    idx = jax.lax.broadcasted_iota(jnp.int32, (1, L), 1)

    for r in range(_ROWS):
        x = x_ref[r]

        y = jnp.mean(x, axis=1, keepdims=True)
        h = jnp.dot(sew1t_ref[...], y, preferred_element_type=jnp.float32)
        h = jnp.maximum(h + seb1_ref[...], 0.0)
        s = jnp.dot(sew2t_ref[...], h, preferred_element_type=jnp.float32)
        s = jax.nn.sigmoid(s + seb2_ref[...])
        gate = s * cas_ref[...]
        shift = cash_ref[...]

        a = jnp.dot(w8_ref[...], x, preferred_element_type=jnp.float32)

        ta = (tac_ref[...] + shr(a[0:1], 8) + shr(a[1:2], 4) + a[2:3]
              + shl(a[3:4], 4) + shl(a[4:5], 8))
        ta = ta - jnp.where(idx < 4, a[5:6], 0.0)
        ta = ta - jnp.where(idx >= L - 4, a[6:7], 0.0)

        ca = gate * x + shift
        o_ref[r] = ca * (ta * x) + x


def kernel(x, eps, se_w1, se_b1, se_w2, se_b2, bn_ca_gamma, bn_ca_beta,
           bn_ca_mean, bn_ca_var, ta_w1, ta_b1, ta_w2, ta_b2, ta_w3, ta_b3,
           ta_w4, ta_b4, bn_ta_gamma, bn_ta_beta, bn_ta_mean, bn_ta_var):
    B, C, L = x.shape
    Cr = se_w1.shape[1]

    bn_scale = bn_ca_gamma * jax.lax.rsqrt(bn_ca_var + eps)
    cas = bn_scale[:, None]
    cash = (bn_ca_beta - bn_ca_mean * bn_scale)[:, None]

    bnt_scale = bn_ta_gamma * jax.lax.rsqrt(bn_ta_var + eps)
    w4e = ta_w4 * bnt_scale
    b4e = ta_b4 * bnt_scale + bn_ta_beta - bn_ta_mean * bnt_scale

    w2t = jnp.transpose(ta_w2, (2, 0, 1))
    w3t = jnp.transpose(ta_w3, (2, 0, 1))
    v = jnp.einsum('c,jcd->jd', w4e[0], w3t)
    m = jnp.einsum('jc,icd->jid', v, w2t)
    u = jnp.stack([m[0, 0], m[0, 1] + m[1, 0],
                   m[0, 2] + m[1, 1] + m[2, 0],
                   m[1, 2] + m[2, 1], m[2, 2]])
    w8 = jnp.concatenate([u @ ta_w1,
                          (m[0, 2] @ ta_w1)[None],
                          (m[2, 0] @ ta_w1)[None],
                          jnp.zeros((1, C), jnp.float32)], axis=0)

    t = jnp.arange(L)
    const = jnp.full((L,), b4e[0] + w4e[0] @ ta_b3, jnp.float32)
    for j in range(3):
        o = 4 * (j - 1)
        const += jnp.where((t + o >= 0) & (t + o < L), v[j] @ ta_b2, 0.0)
    for sft in range(5):
        o = 4 * (sft - 2)
        const += jnp.where((t + o >= 0) & (t + o < L), u[sft] @ ta_b1, 0.0)
    const -= jnp.where(t < 4, m[0, 2] @ ta_b1, 0.0)
    const -= jnp.where(t >= L - 4, m[2, 0] @ ta_b1, 0.0)
    tac = const[None, :]

    sew1t = se_w1.T
    sew2t = se_w2.T
    seb1 = se_b1[:, None]
    seb2 = se_b2[:, None]

    nbytes = x.size * x.dtype.itemsize
    cost = pl.CostEstimate(
        flops=2 * B * 8 * C * L + 7 * B * C * L,
        transcendentals=B * C,
        bytes_accessed=2 * nbytes,
    )
    return pl.pallas_call(
        _bam_fused_kernel,
        out_shape=jax.ShapeDtypeStruct((B, C, L), x.dtype),
        grid=(B // _ROWS,),
        in_specs=[
            pl.BlockSpec((_ROWS, C, L), lambda b: (b, 0, 0)),
            pl.BlockSpec((Cr, C), lambda b: (0, 0)),
            pl.BlockSpec((Cr, 1), lambda b: (0, 0)),
            pl.BlockSpec((C, Cr), lambda b: (0, 0)),
            pl.BlockSpec((C, 1), lambda b: (0, 0)),
            pl.BlockSpec((C, 1), lambda b: (0, 0)),
            pl.BlockSpec((C, 1), lambda b: (0, 0)),
            pl.BlockSpec((8, C), lambda b: (0, 0)),
            pl.BlockSpec((1, L), lambda b: (0, 0)),
        ],
        out_specs=pl.BlockSpec((_ROWS, C, L), lambda b: (b, 0, 0)),
        compiler_params=pltpu.CompilerParams(
            dimension_semantics=("parallel",)),
        cost_estimate=cost,
    )(x, sew1t, seb1, sew2t, seb2, cas, cash, w8, tac)


# speedup vs baseline: 2.2861x; 1.0434x over previous
import jax
import jax.numpy as jnp
from jax.experimental import pallas as pl
from jax.experimental.pallas import tpu as pltpu


_ROWS = 4


def _bam_fused_kernel(x_ref, sew1t_ref, seb1_ref, sew2t_ref, seb2_ref,
                      cas_ref, cash_ref, w8_ref, tac_ref, o_ref):
    L = x_ref.shape[2]

    def shr(r, n):
        return jnp.concatenate(
            [jnp.zeros((1, n), jnp.float32), r[:, :L - n]], axis=1)

    def shl(r, n):
        return jnp.concatenate(
            [r[:, n:], jnp.zeros((1, n), jnp.float32)], axis=1)

    idx = jax.lax.broadcasted_iota(jnp.int32, (1, L), 1)

    for r in range(_ROWS):
        x = x_ref[r]

        y = jnp.mean(x, axis=1, keepdims=True)
        h = jnp.dot(sew1t_ref[...], y, preferred_element_type=jnp.float32)
        h = jnp.maximum(h + seb1_ref[...], 0.0)
        s = jnp.dot(sew2t_ref[...], h, preferred_element_type=jnp.float32)
        s = jax.nn.sigmoid(s + seb2_ref[...])
        gate = s * cas_ref[...]
        shift = cash_ref[...]

        a = jnp.dot(w8_ref[...], x, preferred_element_type=jnp.float32)

        ta = (tac_ref[...] + shr(a[0:1], 8) + shr(a[1:2], 4) + a[2:3]
              + shl(a[3:4], 4) + shl(a[4:5], 8))
        ta = ta - jnp.where(idx < 4, a[5:6], 0.0)
        ta = ta - jnp.where(idx >= L - 4, a[6:7], 0.0)

        ca = gate * x + shift
        o_ref[r] = ca * (ta * x) + x


def kernel(x, eps, se_w1, se_b1, se_w2, se_b2, bn_ca_gamma, bn_ca_beta,
           bn_ca_mean, bn_ca_var, ta_w1, ta_b1, ta_w2, ta_b2, ta_w3, ta_b3,
           ta_w4, ta_b4, bn_ta_gamma, bn_ta_beta, bn_ta_mean, bn_ta_var):
    B, C, L = x.shape
    Cr = se_w1.shape[1]

    bn_scale = bn_ca_gamma * jax.lax.rsqrt(bn_ca_var + eps)
    cas = bn_scale[:, None]
    cash = (bn_ca_beta - bn_ca_mean * bn_scale)[:, None]

    bnt_scale = bn_ta_gamma * jax.lax.rsqrt(bn_ta_var + eps)
    w4e = ta_w4 * bnt_scale
    b4e = ta_b4 * bnt_scale + bn_ta_beta - bn_ta_mean * bnt_scale

    w2t = jnp.transpose(ta_w2, (2, 0, 1))
    w3t = jnp.transpose(ta_w3, (2, 0, 1))
    v = jnp.einsum('c,jcd->jd', w4e[0], w3t)
    m = jnp.einsum('jc,icd->jid', v, w2t)
    u = jnp.stack([m[0, 0], m[0, 1] + m[1, 0],
                   m[0, 2] + m[1, 1] + m[2, 0],
                   m[1, 2] + m[2, 1], m[2, 2]])
    w8 = jnp.concatenate([u @ ta_w1,
                          (m[0, 2] @ ta_w1)[None],
                          (m[2, 0] @ ta_w1)[None],
                          jnp.zeros((1, C), jnp.float32)], axis=0)

    t = jnp.arange(L)
    const = jnp.full((L,), b4e[0] + w4e[0] @ ta_b3, jnp.float32)
    for j in range(3):
        o = 4 * (j - 1)
        const += jnp.where((t + o >= 0) & (t + o < L), v[j] @ ta_b2, 0.0)
    for sft in range(5):
        o = 4 * (sft - 2)
        const += jnp.where((t + o >= 0) & (t + o < L), u[sft] @ ta_b1, 0.0)
    const -= jnp.where(t < 4, m[0, 2] @ ta_b1, 0.0)
    const -= jnp.where(t >= L - 4, m[2, 0] @ ta_b1, 0.0)
    tac = const[None, :]

    sew1t = se_w1.T
    sew2t = se_w2.T
    seb1 = se_b1[:, None]
    seb2 = se_b2[:, None]

    nbytes = x.size * x.dtype.itemsize
    cost = pl.CostEstimate(
        flops=2 * B * 8 * C * L + 7 * B * C * L,
        transcendentals=B * C,
        bytes_accessed=2 * nbytes,
    )
    return pl.pallas_call(
        _bam_fused_kernel,
        out_shape=jax.ShapeDtypeStruct((B, C, L), x.dtype),
        grid=(B // _ROWS,),
        in_specs=[
            pl.BlockSpec((_ROWS, C, L), lambda b: (b, 0, 0)),
            pl.BlockSpec((Cr, C), lambda b: (0, 0)),
            pl.BlockSpec((Cr, 1), lambda b: (0, 0)),
            pl.BlockSpec((C, Cr), lambda b: (0, 0)),
            pl.BlockSpec((C, 1), lambda b: (0, 0)),
            pl.BlockSpec((C, 1), lambda b: (0, 0)),
            pl.BlockSpec((C, 1), lambda b: (0, 0)),
            pl.BlockSpec((8, C), lambda b: (0, 0)),
            pl.BlockSpec((1, L), lambda b: (0, 0)),
        ],
        out_specs=pl.BlockSpec((_ROWS, C, L), lambda b: (b, 0, 0)),
        compiler_params=pltpu.CompilerParams(
            dimension_semantics=("parallel",)),
        cost_estimate=cost,
    )(x, sew1t, seb1, sew2t, seb2, cas, cash, w8, tac)
